# Initial kernel scaffold; baseline (speedup 1.0000x reference)
import jax
import jax.numpy as jnp
from jax.experimental import pallas as pl


def kernel(pred_logits, pred_boxes, tgt_labels, tgt_boxes):
    raise NotImplementedError("write your pallas kernel here")


# speedup vs baseline: 6.5878x; 6.5878x over previous
import functools

import jax
import jax.numpy as jnp
from jax.experimental import pallas as pl
from jax.experimental.pallas import tpu as pltpu

B, NQ, NC, NT = 4, 2000, 91, 300
COST_CLASS, COST_BBOX, COST_GIOU = 1.0, 5.0, 2.0


def _matcher_body(logitsT_ref, pboxT_ref, tlab_ref, tbox_ref,
                  src_ref, tgt_ref, probT_ref, costT_ref):
    x = logitsT_ref[0]
    x_max = jnp.max(x, axis=0, keepdims=True)
    e = jnp.exp(x - x_max)
    s = jnp.sum(e, axis=0, keepdims=True)
    probT_ref[...] = e / s

    def cls_row(j, _):
        cid = tlab_ref[0, 0, j]
        costT_ref[pl.ds(j, 1), :] = -probT_ref[pl.ds(cid, 1), :]
        return 0
    jax.lax.fori_loop(0, NT, cls_row, 0)

    pb = pboxT_ref[0]
    tb = tbox_ref[0]
    pcx, pcy, pw, ph = (pb[0:1, :], pb[1:2, :], pb[2:3, :], pb[3:4, :])
    tcx, tcy, tw, th = (tb[:, 0:1], tb[:, 1:2], tb[:, 2:3], tb[:, 3:4])

    bb = jnp.abs(pcx - tcx)
    bb = bb + jnp.abs(pcy - tcy)
    bb = bb + jnp.abs(pw - tw)
    bb = bb + jnp.abs(ph - th)

    px1, py1 = pcx - 0.5 * pw, pcy - 0.5 * ph
    px2, py2 = pcx + 0.5 * pw, pcy + 0.5 * ph
    tx1, ty1 = tcx - 0.5 * tw, tcy - 0.5 * th
    tx2, ty2 = tcx + 0.5 * tw, tcy + 0.5 * th

    area_p = jnp.maximum(px2 - px1, 0.0) * jnp.maximum(py2 - py1, 0.0)
    area_t = jnp.maximum(tx2 - tx1, 0.0) * jnp.maximum(ty2 - ty1, 0.0)

    iw = jnp.maximum(jnp.minimum(px2, tx2) - jnp.maximum(px1, tx1), 0.0)
    ih = jnp.maximum(jnp.minimum(py2, ty2) - jnp.maximum(py1, ty1), 0.0)
    inter = iw * ih
    union = area_p + area_t - inter
    iou = inter / jnp.maximum(union, 1e-06)

    ew = jnp.maximum(jnp.maximum(px2, tx2) - jnp.minimum(px1, tx1), 0.0)
    eh = jnp.maximum(jnp.maximum(py2, ty2) - jnp.minimum(py1, ty1), 0.0)
    enc = ew * eh
    giou = iou - (enc - union) / jnp.maximum(enc, 1e-06)

    cc = costT_ref[...]
    costT_ref[...] = COST_CLASS * cc + COST_BBOX * bb + COST_GIOU * (-giou)

    jidx = jax.lax.broadcasted_iota(jnp.int32, (NT, NQ), 0)
    qidx = jax.lax.broadcasted_iota(jnp.int32, (NT, NQ), 1)
    lane = jax.lax.broadcasted_iota(jnp.int32, (1, 1, NT), 2)

    def step(i, _):
        c = costT_ref[...]
        m = jnp.min(c)
        eq = c == m
        r = jnp.min(jnp.where(eq, qidx, NQ))
        t = jnp.min(jnp.where(eq & (qidx == r), jidx, NT))
        src_ref[...] = jnp.where(lane == i, r, src_ref[...])
        tgt_ref[...] = jnp.where(lane == i, t, tgt_ref[...])
        costT_ref[...] = jnp.where((qidx == r) | (jidx == t), jnp.inf, c)
        return 0
    jax.lax.fori_loop(0, NT, step, 0)


@jax.jit
def kernel(pred_logits, pred_boxes, tgt_labels, tgt_boxes):
    logitsT = jnp.transpose(pred_logits, (0, 2, 1))
    pboxT = jnp.transpose(pred_boxes, (0, 2, 1))
    tlab3 = tgt_labels.reshape(B, 1, NT)

    grid_spec = pltpu.PrefetchScalarGridSpec(
        num_scalar_prefetch=0,
        grid=(B,),
        in_specs=[
            pl.BlockSpec((1, NC, NQ), lambda b: (b, 0, 0)),
            pl.BlockSpec((1, 4, NQ), lambda b: (b, 0, 0)),
            pl.BlockSpec((1, 1, NT), lambda b: (b, 0, 0),
                         memory_space=pltpu.SMEM),
            pl.BlockSpec((1, NT, 4), lambda b: (b, 0, 0)),
        ],
        out_specs=[
            pl.BlockSpec((1, 1, NT), lambda b: (b, 0, 0)),
            pl.BlockSpec((1, 1, NT), lambda b: (b, 0, 0)),
        ],
        scratch_shapes=[
            pltpu.VMEM((NC, NQ), jnp.float32),
            pltpu.VMEM((NT, NQ), jnp.float32),
        ],
    )
    src, tgt = pl.pallas_call(
        _matcher_body,
        grid_spec=grid_spec,
        out_shape=[
            jax.ShapeDtypeStruct((B, 1, NT), jnp.int32),
            jax.ShapeDtypeStruct((B, 1, NT), jnp.int32),
        ],
    )(logitsT, pboxT, tlab3, tgt_boxes)
    return src.reshape(B, NT), tgt.reshape(B, NT)
